# baseline (speedup 1.0000x reference)
import jax
import jax.numpy as jnp
from jax import lax
from jax.experimental import pallas as pl
from jax.experimental.pallas import tpu as pltpu
from jax.experimental.pallas import tpu_sc as plsc

BATCH = 4096
P = 200
EMBED_DIM = 128
N = BATCH * P
NC = 2
NS = 16
NW = NC * NS
K = 128
B_PER_W = N // NW
N_CHUNKS = B_PER_W // K
NBUF = 4
L = 16


def _emb_body(ids_hbm, table_hbm, out_hbm, idx_all, tbl_flat, rows, ssems):
    c = lax.axis_index("c")
    s = lax.axis_index("s")
    wid = s * NC + c
    chunk0 = wid * N_CHUNKS

    pltpu.sync_copy(table_hbm, tbl_flat)
    pltpu.sync_copy(ids_hbm.at[pl.ds(chunk0, N_CHUNKS)], idx_all)

    def s_copy(j, b):
        return pltpu.make_async_copy(
            rows[b],
            out_hbm.at[pl.ds((chunk0 + j) * K * EMBED_DIM, K * EMBED_DIM)],
            ssems[b])

    iota16 = lax.iota(jnp.int32, L)

    def build(j, b):
        @plsc.parallel_loop(0, K // L)
        def grp(g):
            srcb = idx_all[j, pl.ds(g * L, L)] * EMBED_DIM
            dstb = (g * L + iota16) * EMBED_DIM

            @plsc.parallel_loop(0, EMBED_DIM, unroll=32)
            def col(c):
                colv = (iota16 + c) & (EMBED_DIM - 1)
                vals = plsc.load_gather(tbl_flat, [srcb + colv])
                plsc.store_scatter(rows[b], [dstb + colv], vals)

    @pl.loop(0, N_CHUNKS, step=NBUF)
    def ring(outer):
        for b in range(NBUF):
            j = outer + b

            @pl.when(j >= NBUF)
            def _():
                s_copy(j - NBUF, b).wait()

            build(j, b)
            s_copy(j, b).start()

    for b in range(NBUF):
        s_copy(N_CHUNKS - NBUF + b, b).wait()


@jax.jit
def _emb_lookup(ids_2d, table):
    mesh = plsc.VectorSubcoreMesh(core_axis_name="c", subcore_axis_name="s")
    return pl.kernel(
        _emb_body,
        out_type=jax.ShapeDtypeStruct((N * EMBED_DIM,), jnp.float32),
        mesh=mesh,
        compiler_params=pltpu.CompilerParams(needs_layout_passes=False),
        scratch_types=[
            pltpu.VMEM((N_CHUNKS, K), jnp.int32),
            pltpu.VMEM((3 * EMBED_DIM,), jnp.float32),
            [pltpu.VMEM((K * EMBED_DIM,), jnp.float32)] * NBUF,
            [pltpu.SemaphoreType.DMA] * NBUF,
        ],
    )(ids_2d, table)


def kernel(perturbation_ids, table):
    table = table.reshape(3 * EMBED_DIM)
    ids_2d = perturbation_ids.astype(jnp.int32).reshape(N // K, K)
    out = _emb_lookup(ids_2d, table)
    return out.reshape(BATCH, P, EMBED_DIM)


# speedup vs baseline: 1.0811x; 1.0811x over previous
import jax
import jax.numpy as jnp
from jax import lax
from jax.experimental import pallas as pl
from jax.experimental.pallas import tpu as pltpu
from jax.experimental.pallas import tpu_sc as plsc

BATCH = 4096
P = 200
EMBED_DIM = 128
N = BATCH * P
NC = 2
NS = 16
NW = NC * NS
K = 128
B_PER_W = N // NW
N_CHUNKS = B_PER_W // K
NBUF = 5
L = 16


def _emb_body(ids_hbm, table_hbm, out_hbm, idx_all, tbl_flat, rows, ssems):
    c = lax.axis_index("c")
    s = lax.axis_index("s")
    wid = s * NC + c
    chunk0 = wid * N_CHUNKS

    pltpu.sync_copy(table_hbm, tbl_flat)
    pltpu.sync_copy(ids_hbm.at[pl.ds(chunk0, N_CHUNKS)], idx_all)

    def s_copy(j, b):
        return pltpu.make_async_copy(
            rows[b],
            out_hbm.at[pl.ds((chunk0 + j) * K * EMBED_DIM, K * EMBED_DIM)],
            ssems[b])

    iota16 = lax.iota(jnp.int32, L)

    def build(j, b):
        @plsc.parallel_loop(0, K // L)
        def grp(g):
            srcb = idx_all[j, pl.ds(g * L, L)] * EMBED_DIM
            dstb = (g * L + iota16) * EMBED_DIM

            @plsc.parallel_loop(0, EMBED_DIM, unroll=16)
            def col(c):
                colv = (iota16 + c) & (EMBED_DIM - 1)
                vals = plsc.load_gather(tbl_flat, [srcb + colv])
                plsc.store_scatter(rows[b], [dstb + colv], vals)

    @pl.loop(0, N_CHUNKS, step=NBUF)
    def ring(outer):
        for b in range(NBUF):
            j = outer + b

            @pl.when(j >= NBUF)
            def _():
                s_copy(j - NBUF, b).wait()

            build(j, b)
            s_copy(j, b).start()

    for b in range(NBUF):
        s_copy(N_CHUNKS - NBUF + b, b).wait()


@jax.jit
def _emb_lookup(ids_2d, table):
    mesh = plsc.VectorSubcoreMesh(core_axis_name="c", subcore_axis_name="s")
    return pl.kernel(
        _emb_body,
        out_type=jax.ShapeDtypeStruct((N * EMBED_DIM,), jnp.float32),
        mesh=mesh,
        compiler_params=pltpu.CompilerParams(needs_layout_passes=False),
        scratch_types=[
            pltpu.VMEM((N_CHUNKS, K), jnp.int32),
            pltpu.VMEM((3 * EMBED_DIM,), jnp.float32),
            [pltpu.VMEM((K * EMBED_DIM,), jnp.float32)] * NBUF,
            [pltpu.SemaphoreType.DMA] * NBUF,
        ],
    )(ids_2d, table)


def kernel(perturbation_ids, table):
    table = table.reshape(3 * EMBED_DIM)
    ids_2d = perturbation_ids.astype(jnp.int32).reshape(N // K, K)
    out = _emb_lookup(ids_2d, table)
    return out.reshape(BATCH, P, EMBED_DIM)


# speedup vs baseline: 1.2683x; 1.1731x over previous
import jax
import jax.numpy as jnp
from jax import lax
from jax.experimental import pallas as pl
from jax.experimental.pallas import tpu as pltpu
from jax.experimental.pallas import tpu_sc as plsc

BATCH = 4096
P = 200
EMBED_DIM = 128
N = BATCH * P
NC = 2
NS = 16
NW = NC * NS
K = 128
B_PER_W = N // NW
N_CHUNKS = B_PER_W // K
NBUF = 4
L = 16


def _emb_body(ids_hbm, table_hbm, out_hbm, idx_all, tbl_flat, rows, ssems):
    c = lax.axis_index("c")
    s = lax.axis_index("s")
    wid = s * NC + c
    chunk0 = wid * N_CHUNKS

    pltpu.sync_copy(table_hbm, tbl_flat)
    pltpu.sync_copy(ids_hbm.at[pl.ds(chunk0, N_CHUNKS)], idx_all)

    def s_copy(j, b):
        return pltpu.make_async_copy(
            rows[b],
            out_hbm.at[pl.ds((chunk0 + j) * K * EMBED_DIM, K * EMBED_DIM)],
            ssems[b])

    iota16 = lax.iota(jnp.int32, L)

    def build(j, b):
        @plsc.parallel_loop(0, K // L)
        def grp(g):
            srcb = idx_all[j, pl.ds(g * L, L)] * EMBED_DIM
            dstb = (g * L + iota16) * EMBED_DIM
            sv = srcb + iota16
            dv = dstb + iota16

            @plsc.parallel_loop(0, EMBED_DIM - L, unroll=16)
            def col(c):
                vals = plsc.load_gather(tbl_flat, [sv + c])
                plsc.store_scatter(rows[b], [dv + c], vals)

            @plsc.parallel_loop(EMBED_DIM - L, EMBED_DIM, unroll=16)
            def tail(c):
                colv = (iota16 + c) & (EMBED_DIM - 1)
                vals = plsc.load_gather(tbl_flat, [srcb + colv])
                plsc.store_scatter(rows[b], [dstb + colv], vals)

    @pl.loop(0, N_CHUNKS, step=NBUF)
    def ring(outer):
        for b in range(NBUF):
            j = outer + b

            @pl.when(j >= NBUF)
            def _():
                s_copy(j - NBUF, b).wait()

            build(j, b)
            s_copy(j, b).start()

    for b in range(NBUF):
        s_copy(N_CHUNKS - NBUF + b, b).wait()


@jax.jit
def _emb_lookup(ids_2d, table):
    mesh = plsc.VectorSubcoreMesh(core_axis_name="c", subcore_axis_name="s")
    return pl.kernel(
        _emb_body,
        out_type=jax.ShapeDtypeStruct((N * EMBED_DIM,), jnp.float32),
        mesh=mesh,
        compiler_params=pltpu.CompilerParams(needs_layout_passes=False),
        scratch_types=[
            pltpu.VMEM((N_CHUNKS, K), jnp.int32),
            pltpu.VMEM((3 * EMBED_DIM,), jnp.float32),
            [pltpu.VMEM((K * EMBED_DIM,), jnp.float32)] * NBUF,
            [pltpu.SemaphoreType.DMA] * NBUF,
        ],
    )(ids_2d, table)


def kernel(perturbation_ids, table):
    table = table.reshape(3 * EMBED_DIM)
    ids_2d = perturbation_ids.astype(jnp.int32).reshape(N // K, K)
    out = _emb_lookup(ids_2d, table)
    return out.reshape(BATCH, P, EMBED_DIM)
